# baseline (speedup 1.0000x reference)
import functools

import jax
import jax.numpy as jnp
from jax.experimental import pallas as pl
from jax.experimental.pallas import tpu as pltpu

N, D, H, E, K = 32768, 768, 64, 64, 2
CHUNK = 2048
P = 2
GRID = N // (CHUNK * P)


def _reduce_body(x_ref, w_ref, out_ref, acc_ref):
    i = pl.program_id(1)

    @pl.when(i == 0)
    def _init():
        acc_ref[...] = jnp.zeros_like(acc_ref)

    y = (x_ref[...] * w_ref[...]).reshape(CHUNK // 8, 8, D)
    acc_ref[...] += jnp.sum(y, axis=0)

    @pl.when(i == GRID - 1)
    def _finish():
        out_ref[...] = acc_ref[...]


def _gate_body(part_ref, win_ref, wlin_ref, idx_ref, p_ref):
    v = jnp.sum(part_ref[...], axis=0, keepdims=True)
    h = jax.lax.dot_general(
        v, win_ref[...], (((1,), (1,)), ((), ())),
        preferred_element_type=jnp.float32)
    o = jax.lax.dot_general(
        h, wlin_ref[...], (((1,), (1,)), ((), ())),
        preferred_element_type=jnp.float32)

    iota = jax.lax.broadcasted_iota(jnp.int32, (1, E), 1)
    m1 = jnp.max(o)
    i1 = jnp.min(jnp.where(o == m1, iota, E))
    masked = jnp.where(iota == i1, -jnp.inf, o)
    m2 = jnp.max(masked)
    i2 = jnp.min(jnp.where(masked == m2, iota, E))
    e = jnp.exp(m2 - m1)
    p1 = 1.0 / (1.0 + e)

    pos = jax.lax.broadcasted_iota(jnp.int32, (1, 2), 1)
    idx_ref[...] = jnp.where(pos == 0, i1, i2)
    p_ref[...] = jnp.where(pos == 0, p1, 1.0 - p1)


@functools.partial(jax.jit, static_argnames=("interpret",))
def kernel(x, W_in, W_lin, W_out, interpret=False):
    partials = pl.pallas_call(
        _reduce_body,
        grid=(P, GRID),
        in_specs=[
            pl.BlockSpec((CHUNK, D), lambda c, i: (c * GRID + i, 0)),
            pl.BlockSpec((CHUNK, 1), lambda c, i: (c * GRID + i, 0)),
        ],
        out_specs=pl.BlockSpec((8, D), lambda c, i: (c, 0)),
        out_shape=jax.ShapeDtypeStruct((8 * P, D), jnp.float32),
        scratch_shapes=[pltpu.VMEM((8, D), jnp.float32)],
        compiler_params=pltpu.CompilerParams(
            dimension_semantics=("parallel", "arbitrary")),
        interpret=interpret,
    )(x, W_out.reshape(N, 1))

    idx2, p2 = pl.pallas_call(
        _gate_body,
        out_shape=[
            jax.ShapeDtypeStruct((1, 2), jnp.int32),
            jax.ShapeDtypeStruct((1, 2), jnp.float32),
        ],
        interpret=interpret,
    )(partials, W_in, W_lin)
    return idx2.reshape(-1), p2.reshape(-1)


# speedup vs baseline: 1.3832x; 1.3832x over previous
import functools

import jax
import jax.numpy as jnp
from jax.experimental import pallas as pl
from jax.experimental.pallas import tpu as pltpu

N, D, H, E, K = 32768, 768, 64, 64, 2
CHUNK = 2048
NBUF = 4
GRID = N // CHUNK


def _gate_body(x_hbm, w_ref, win_ref, wlin_ref, idx_ref, p_ref,
               buf_ref, sems):
    def copy_in(slot, step):
        return pltpu.make_async_copy(
            x_hbm.at[pl.ds(step * CHUNK, CHUNK), :],
            buf_ref.at[slot],
            sems.at[slot])

    for s in range(NBUF - 1):
        copy_in(s, s).start()

    def body(i, acc):
        slot = jax.lax.rem(i, NBUF)
        nxt = i + NBUF - 1

        @pl.when(nxt < GRID)
        def _():
            copy_in(jax.lax.rem(nxt, NBUF), nxt).start()

        copy_in(slot, i).wait()
        w = w_ref[i]
        return acc + jax.lax.dot_general(
            w, buf_ref[slot], (((1,), (0,)), ((), ())),
            preferred_element_type=jnp.float32)

    acc = jax.lax.fori_loop(0, GRID, body, jnp.zeros((1, D), jnp.float32))

    h = jax.lax.dot_general(
        acc, win_ref[...], (((1,), (1,)), ((), ())),
        preferred_element_type=jnp.float32)
    o = jax.lax.dot_general(
        h, wlin_ref[...], (((1,), (1,)), ((), ())),
        preferred_element_type=jnp.float32)

    iota = jax.lax.broadcasted_iota(jnp.int32, (1, E), 1)
    m1 = jnp.max(o)
    i1 = jnp.min(jnp.where(o == m1, iota, E))
    masked = jnp.where(iota == i1, -jnp.inf, o)
    m2 = jnp.max(masked)
    i2 = jnp.min(jnp.where(masked == m2, iota, E))
    e = jnp.exp(m2 - m1)
    p1 = 1.0 / (1.0 + e)

    pos = jax.lax.broadcasted_iota(jnp.int32, (1, 2), 1)
    idx_ref[...] = jnp.where(pos == 0, i1, i2)
    p_ref[...] = jnp.where(pos == 0, p1, 1.0 - p1)


@functools.partial(jax.jit, static_argnames=("interpret",))
def kernel(x, W_in, W_lin, W_out, interpret=False):
    idx2, p2 = pl.pallas_call(
        _gate_body,
        in_specs=[
            pl.BlockSpec(memory_space=pl.ANY),
            pl.BlockSpec(memory_space=pltpu.MemorySpace.VMEM),
            pl.BlockSpec(memory_space=pltpu.MemorySpace.VMEM),
            pl.BlockSpec(memory_space=pltpu.MemorySpace.VMEM),
        ],
        out_specs=[
            pl.BlockSpec(memory_space=pltpu.MemorySpace.VMEM),
            pl.BlockSpec(memory_space=pltpu.MemorySpace.VMEM),
        ],
        out_shape=[
            jax.ShapeDtypeStruct((1, 2), jnp.int32),
            jax.ShapeDtypeStruct((1, 2), jnp.float32),
        ],
        scratch_shapes=[
            pltpu.VMEM((NBUF, CHUNK, D), jnp.float32),
            pltpu.SemaphoreType.DMA((NBUF,)),
        ],
        interpret=interpret,
    )(x, W_out.reshape(GRID, 1, CHUNK), W_in, W_lin)
    return idx2.reshape(-1), p2.reshape(-1)
